# Initial kernel scaffold; baseline (speedup 1.0000x reference)
import jax
import jax.numpy as jnp
from jax.experimental import pallas as pl


def kernel(x, edge_index, W1_l, b1, W1_r, W2_l, b2, W2_r):
    raise NotImplementedError("write your pallas kernel here")


# speedup vs baseline: 5.3370x; 5.3370x over previous
import functools

import jax
import jax.numpy as jnp
from jax import lax
from jax.experimental import pallas as pl
from jax.experimental.pallas import tpu as pltpu
from jax.experimental.pallas import tpu_sc as plsc

N_NODES = 10000
N_EDGES = 320000
IN_DIM = 128
HID_DIM = 256
OUT_DIM = 64

NC, NS = 2, 16
NW = NC * NS
EPT = N_EDGES // NW
CHUNK = 80
NCHUNK = EPT // CHUNK


ROWS_TILE = N_NODES // NS
N_PAD = 10240
CNT_TILE = N_PAD // NS


def _zero_spmem_slice(buf, spm, sid, nbuf_rows):
    base = sid * ROWS_TILE
    nfull = ROWS_TILE // nbuf_rows
    rem = ROWS_TILE % nbuf_rows
    for k in range(nfull):
        pltpu.sync_copy(buf, spm.at[pl.ds(base + k * nbuf_rows, nbuf_rows)])
    if rem:
        pltpu.sync_copy(buf.at[pl.ds(0, rem)],
                        spm.at[pl.ds(base + ROWS_TILE - rem, rem)])


def _seg_body(D, table_hbm, src_hbm, dst_hbm, acc_out, srcb, dstb, rows,
              acc_sh, sem):
    cid = lax.axis_index("c")
    sid = lax.axis_index("s")
    w = cid * NS + sid
    ncol = D // 16
    zero16 = jnp.zeros((16,), jnp.float32)

    def z_rows(k, _):
        rows[k // ncol, pl.ds((k % ncol) * 16, 16)] = zero16
        return 0
    lax.fori_loop(0, CHUNK * ncol, z_rows, 0)
    _zero_spmem_slice(rows, acc_sh, sid, CHUNK)

    plsc.subcore_barrier()

    def step(j, _):
        off = pl.multiple_of(w * EPT + j * CHUNK, 8)
        pltpu.sync_copy(src_hbm.at[pl.ds(off, CHUNK)], srcb)
        pltpu.sync_copy(dst_hbm.at[pl.ds(off, CHUNK)], dstb)
        pltpu.async_copy(table_hbm.at[srcb], rows, sem).wait()
        pltpu.sync_copy(rows, acc_sh.at[dstb], add=True)
        return 0
    lax.fori_loop(0, NCHUNK, step, 0)

    plsc.subcore_barrier()
    base = sid * ROWS_TILE
    pltpu.sync_copy(acc_sh.at[pl.ds(base, ROWS_TILE)], acc_out.at[cid, sid])


def _cnt_body(dst_hbm, cnt_out, dstb, ones, cnt_sh):
    cid = lax.axis_index("c")
    sid = lax.axis_index("s")
    w = cid * NS + sid
    ncol = IN_DIM // 16
    zero16 = jnp.zeros((16,), jnp.float32)

    def z_rows(k, _):
        ones[k // ncol, pl.ds((k % ncol) * 16, 16)] = zero16
        return 0
    lax.fori_loop(0, CHUNK * ncol, z_rows, 0)
    _zero_spmem_slice(ones, cnt_sh, sid, CHUNK)

    one16 = jnp.ones((16,), jnp.float32)

    def f_rows(k, _):
        ones[k // ncol, pl.ds((k % ncol) * 16, 16)] = one16
        return 0
    lax.fori_loop(0, CHUNK * ncol, f_rows, 0)

    plsc.subcore_barrier()

    def step(j, _):
        off = pl.multiple_of(w * EPT + j * CHUNK, 8)
        pltpu.sync_copy(dst_hbm.at[pl.ds(off, CHUNK)], dstb)
        pltpu.sync_copy(ones, cnt_sh.at[dstb], add=True)
        return 0
    lax.fori_loop(0, NCHUNK, step, 0)

    plsc.subcore_barrier()
    base = sid * ROWS_TILE
    pltpu.sync_copy(cnt_sh.at[pl.ds(base, ROWS_TILE)], cnt_out.at[cid, sid])


def _make_seg_kernel(D):
    mesh = plsc.VectorSubcoreMesh(core_axis_name="c", subcore_axis_name="s",
                                  num_cores=NC, num_subcores=NS)
    out = jax.ShapeDtypeStruct((NC, NS, ROWS_TILE, D), jnp.float32)
    scratch = [
        pltpu.VMEM((CHUNK,), jnp.int32),
        pltpu.VMEM((CHUNK,), jnp.int32),
        pltpu.VMEM((CHUNK, D), jnp.float32),
        pltpu.VMEM_SHARED((N_NODES, D), jnp.float32),
        pltpu.SemaphoreType.DMA,
    ]
    return pl.kernel(functools.partial(_seg_body, D),
                     out_type=out, mesh=mesh, scratch_types=scratch)


def _make_cnt_kernel():
    mesh = plsc.VectorSubcoreMesh(core_axis_name="c", subcore_axis_name="s",
                                  num_cores=NC, num_subcores=NS)
    out = jax.ShapeDtypeStruct((NC, NS, ROWS_TILE, IN_DIM), jnp.float32)
    scratch = [
        pltpu.VMEM((CHUNK,), jnp.int32),
        pltpu.VMEM((CHUNK, IN_DIM), jnp.float32),
        pltpu.VMEM_SHARED((N_NODES, IN_DIM), jnp.float32),
    ]
    return pl.kernel(_cnt_body, out_type=out, mesh=mesh,
                     scratch_types=scratch)


_seg_l1 = _make_seg_kernel(IN_DIM)
_seg_l2 = _make_seg_kernel(IN_DIM)
_cnt_k = _make_cnt_kernel()

_TC_R = 1000


def _tc1_body(s1a, s1b, cnta, cntb, x, w1l, b1, w1r, w2l, h_ref, y2_ref):
    cnt = jnp.maximum(cnta[...] + cntb[...], 1.0)
    agg = (s1a[...] + s1b[...]) / cnt
    h = (jnp.dot(agg, w1l[...], preferred_element_type=jnp.float32) + b1[...]
         + jnp.dot(x[...], w1r[...], preferred_element_type=jnp.float32))
    h = jnp.maximum(h, 0.0)
    h_ref[...] = h
    y2_ref[...] = jnp.dot(h, w2l[...], preferred_element_type=jnp.float32)


def _tc2_body(s2a, s2b, cnta, cntb, h, w2r, b2, out_ref):
    cnt = jnp.maximum(cnta[...] + cntb[...], 1.0)
    s2 = (s2a[...] + s2b[...])[:, :OUT_DIM]
    o = (s2 / cnt + b2[...]
         + jnp.dot(h[...], w2r[...], preferred_element_type=jnp.float32))
    m = jnp.max(o, axis=1, keepdims=True)
    lse = jnp.log(jnp.sum(jnp.exp(o - m), axis=1, keepdims=True)) + m
    out_ref[...] = o - lse


def _row_spec(d):
    return pl.BlockSpec((_TC_R, d), lambda i: (i, 0))


def _full_spec(r, c):
    return pl.BlockSpec((r, c), lambda i: (0, 0))


_tc1 = pl.pallas_call(
    _tc1_body,
    grid=(N_NODES // _TC_R,),
    in_specs=[
        _row_spec(IN_DIM), _row_spec(IN_DIM),
        _row_spec(1), _row_spec(1),
        _row_spec(IN_DIM),
        _full_spec(IN_DIM, HID_DIM), _full_spec(1, HID_DIM),
        _full_spec(IN_DIM, HID_DIM), _full_spec(HID_DIM, IN_DIM),
    ],
    out_specs=[_row_spec(HID_DIM), _row_spec(IN_DIM)],
    out_shape=[jax.ShapeDtypeStruct((N_NODES, HID_DIM), jnp.float32),
               jax.ShapeDtypeStruct((N_NODES, IN_DIM), jnp.float32)],
)

_tc2 = pl.pallas_call(
    _tc2_body,
    grid=(N_NODES // _TC_R,),
    in_specs=[
        _row_spec(IN_DIM), _row_spec(IN_DIM),
        _row_spec(1), _row_spec(1),
        _row_spec(HID_DIM),
        _full_spec(HID_DIM, OUT_DIM), _full_spec(1, OUT_DIM),
    ],
    out_specs=_row_spec(OUT_DIM),
    out_shape=jax.ShapeDtypeStruct((N_NODES, OUT_DIM), jnp.float32),
)


def kernel(x, edge_index, W1_l, b1, W1_r, W2_l, b2, W2_r):
    ei = edge_index.astype(jnp.int32)
    src = ei[0]
    dst = ei[1]
    s1 = _seg_l1(x, src, dst).reshape(NC, N_NODES, IN_DIM)
    cnt = _cnt_k(dst).reshape(NC, N_NODES, IN_DIM)[:, :, :1]
    W2_l_pad = jnp.pad(W2_l, ((0, 0), (0, IN_DIM - OUT_DIM)))
    h, y2 = _tc1(s1[0], s1[1], cnt[0], cnt[1], x,
                 W1_l, b1.reshape(1, HID_DIM), W1_r, W2_l_pad)
    s2 = _seg_l2(y2, src, dst).reshape(NC, N_NODES, IN_DIM)
    out = _tc2(s2[0], s2[1], cnt[0], cnt[1], h, W2_r, b2.reshape(1, OUT_DIM))
    return out


# speedup vs baseline: 10.9694x; 2.0553x over previous
import functools

import jax
import jax.numpy as jnp
from jax import lax
from jax.experimental import pallas as pl
from jax.experimental.pallas import tpu as pltpu
from jax.experimental.pallas import tpu_sc as plsc

N_NODES = 10000
N_EDGES = 320000
IN_DIM = 128
HID_DIM = 256
OUT_DIM = 64

NC, NS = 2, 16
NW = NC * NS
EPT = N_EDGES // NW
CHUNK = 100
NCHUNK = EPT // CHUNK
NPHASE = 2
PCHUNK = NCHUNK // NPHASE


ROWS_TILE = N_NODES // NS
N_PAD = 10240
CNT_TILE = N_PAD // NS


def _zero_spmem_slice(buf, spm, sid, nbuf_rows):
    base = sid * ROWS_TILE
    nfull = ROWS_TILE // nbuf_rows
    rem = ROWS_TILE % nbuf_rows
    for k in range(nfull):
        pltpu.sync_copy(buf, spm.at[pl.ds(base + k * nbuf_rows, nbuf_rows)])
    if rem:
        pltpu.sync_copy(buf.at[pl.ds(0, rem)],
                        spm.at[pl.ds(base + ROWS_TILE - rem, rem)])


def _seg_body(D, table_hbm, src_hbm, dst_hbm, acc_out, src_all, dst_all,
              rows0, rows1, acc_sh, sem0, sem1):
    cid = lax.axis_index("c")
    sid = lax.axis_index("s")
    w = cid * NS + sid
    ncol = D // 16
    zero16 = jnp.zeros((16,), jnp.float32)

    def z_rows(k, _):
        rows0[k // ncol, pl.ds((k % ncol) * 16, 16)] = zero16
        return 0
    lax.fori_loop(0, CHUNK * ncol, z_rows, 0)
    _zero_spmem_slice(rows0, acc_sh, sid, CHUNK)

    plsc.subcore_barrier()

    def gather(j, buf, sem):
        return pltpu.async_copy(table_hbm.at[src_all.at[j]], buf, sem)

    def wait_g(buf, sem):
        pltpu.make_async_copy(table_hbm.at[src_all.at[0]], buf, sem).wait()

    def scat(j, buf):
        pltpu.sync_copy(buf, acc_sh.at[dst_all.at[j]], add=True)

    for ph in range(NPHASE):
        pltpu.sync_copy(src_hbm.at[w, ph], src_all)
        pltpu.sync_copy(dst_hbm.at[w, ph], dst_all)
        gather(0, rows0, sem0)

        def step2(i, _):
            j0 = 2 * i
            gather(j0 + 1, rows1, sem1)
            wait_g(rows0, sem0)
            scat(j0, rows0)
            gather(j0 + 2, rows0, sem0)
            wait_g(rows1, sem1)
            scat(j0 + 1, rows1)
            return 0
        lax.fori_loop(0, (PCHUNK - 2) // 2, step2, 0)

        gather(PCHUNK - 1, rows1, sem1)
        wait_g(rows0, sem0)
        scat(PCHUNK - 2, rows0)
        wait_g(rows1, sem1)
        scat(PCHUNK - 1, rows1)

    plsc.subcore_barrier()
    base = sid * ROWS_TILE
    pltpu.sync_copy(acc_sh.at[pl.ds(base, ROWS_TILE)], acc_out.at[cid, sid])


def _cnt_body(dst_hbm, cnt_out, dst_all, ones, cnt_sh):
    cid = lax.axis_index("c")
    sid = lax.axis_index("s")
    w = cid * NS + sid
    ncol = IN_DIM // 16
    zero16 = jnp.zeros((16,), jnp.float32)

    def z_rows(k, _):
        ones[k // ncol, pl.ds((k % ncol) * 16, 16)] = zero16
        return 0
    lax.fori_loop(0, CHUNK * ncol, z_rows, 0)
    _zero_spmem_slice(ones, cnt_sh, sid, CHUNK)

    one16 = jnp.ones((16,), jnp.float32)

    def f_rows(k, _):
        ones[k // ncol, pl.ds((k % ncol) * 16, 16)] = one16
        return 0
    lax.fori_loop(0, CHUNK * ncol, f_rows, 0)

    pltpu.sync_copy(dst_hbm.at[w], dst_all)
    plsc.subcore_barrier()

    for ph in range(NPHASE):
        def step(j, _):
            pltpu.sync_copy(ones, cnt_sh.at[dst_all.at[ph, j]], add=True)
            return 0
        lax.fori_loop(0, PCHUNK, step, 0)

    plsc.subcore_barrier()
    base = sid * ROWS_TILE
    pltpu.sync_copy(cnt_sh.at[pl.ds(base, ROWS_TILE)], cnt_out.at[cid, sid])


def _make_seg_kernel(D):
    mesh = plsc.VectorSubcoreMesh(core_axis_name="c", subcore_axis_name="s",
                                  num_cores=NC, num_subcores=NS)
    out = jax.ShapeDtypeStruct((NC, NS, ROWS_TILE, D), jnp.float32)
    scratch = [
        pltpu.VMEM((PCHUNK, CHUNK), jnp.int32),
        pltpu.VMEM((PCHUNK, CHUNK), jnp.int32),
        pltpu.VMEM((CHUNK, D), jnp.float32),
        pltpu.VMEM((CHUNK, D), jnp.float32),
        pltpu.VMEM_SHARED((N_NODES, D), jnp.float32),
        pltpu.SemaphoreType.DMA,
        pltpu.SemaphoreType.DMA,
    ]
    return pl.kernel(functools.partial(_seg_body, D),
                     out_type=out, mesh=mesh, scratch_types=scratch)


def _make_cnt_kernel():
    mesh = plsc.VectorSubcoreMesh(core_axis_name="c", subcore_axis_name="s",
                                  num_cores=NC, num_subcores=NS)
    out = jax.ShapeDtypeStruct((NC, NS, ROWS_TILE, IN_DIM), jnp.float32)
    scratch = [
        pltpu.VMEM((NPHASE, PCHUNK, CHUNK), jnp.int32),
        pltpu.VMEM((CHUNK, IN_DIM), jnp.float32),
        pltpu.VMEM_SHARED((N_NODES, IN_DIM), jnp.float32),
    ]
    return pl.kernel(_cnt_body, out_type=out, mesh=mesh,
                     scratch_types=scratch)


_seg_l1 = _make_seg_kernel(IN_DIM)
_seg_l2 = _make_seg_kernel(IN_DIM)
_cnt_k = _make_cnt_kernel()

_TC_R = 1000


def _tc1_body(s1a, s1b, cnta, cntb, x, w1l, b1, w1r, w2l, h_ref, y2_ref):
    cnt = jnp.maximum(cnta[...] + cntb[...], 1.0)
    agg = (s1a[...] + s1b[...]) / cnt
    h = (jnp.dot(agg, w1l[...], preferred_element_type=jnp.float32) + b1[...]
         + jnp.dot(x[...], w1r[...], preferred_element_type=jnp.float32))
    h = jnp.maximum(h, 0.0)
    h_ref[...] = h
    y2_ref[...] = jnp.dot(h, w2l[...], preferred_element_type=jnp.float32)


def _tc2_body(s2a, s2b, cnta, cntb, h, w2r, b2, out_ref):
    cnt = jnp.maximum(cnta[...] + cntb[...], 1.0)
    s2 = (s2a[...] + s2b[...])[:, :OUT_DIM]
    o = (s2 / cnt + b2[...]
         + jnp.dot(h[...], w2r[...], preferred_element_type=jnp.float32))
    m = jnp.max(o, axis=1, keepdims=True)
    lse = jnp.log(jnp.sum(jnp.exp(o - m), axis=1, keepdims=True)) + m
    out_ref[...] = o - lse


def _row_spec(d):
    return pl.BlockSpec((_TC_R, d), lambda i: (i, 0))


def _full_spec(r, c):
    return pl.BlockSpec((r, c), lambda i: (0, 0))


_tc1 = pl.pallas_call(
    _tc1_body,
    grid=(N_NODES // _TC_R,),
    in_specs=[
        _row_spec(IN_DIM), _row_spec(IN_DIM),
        _row_spec(1), _row_spec(1),
        _row_spec(IN_DIM),
        _full_spec(IN_DIM, HID_DIM), _full_spec(1, HID_DIM),
        _full_spec(IN_DIM, HID_DIM), _full_spec(HID_DIM, IN_DIM),
    ],
    out_specs=[_row_spec(HID_DIM), _row_spec(IN_DIM)],
    out_shape=[jax.ShapeDtypeStruct((N_NODES, HID_DIM), jnp.float32),
               jax.ShapeDtypeStruct((N_NODES, IN_DIM), jnp.float32)],
)

_tc2 = pl.pallas_call(
    _tc2_body,
    grid=(N_NODES // _TC_R,),
    in_specs=[
        _row_spec(IN_DIM), _row_spec(IN_DIM),
        _row_spec(1), _row_spec(1),
        _row_spec(HID_DIM),
        _full_spec(HID_DIM, OUT_DIM), _full_spec(1, OUT_DIM),
    ],
    out_specs=_row_spec(OUT_DIM),
    out_shape=jax.ShapeDtypeStruct((N_NODES, OUT_DIM), jnp.float32),
)


def kernel(x, edge_index, W1_l, b1, W1_r, W2_l, b2, W2_r):
    ei = edge_index.astype(jnp.int32)
    src = ei[0].reshape(NW, NPHASE, PCHUNK, CHUNK)
    dst = ei[1].reshape(NW, NPHASE, PCHUNK, CHUNK)
    s1 = _seg_l1(x, src, dst).reshape(NC, N_NODES, IN_DIM)
    cnt = _cnt_k(dst).reshape(NC, N_NODES, IN_DIM)[:, :, :1]
    W2_l_pad = jnp.pad(W2_l, ((0, 0), (0, IN_DIM - OUT_DIM)))
    h, y2 = _tc1(s1[0], s1[1], cnt[0], cnt[1], x,
                 W1_l, b1.reshape(1, HID_DIM), W1_r, W2_l_pad)
    s2 = _seg_l2(y2, src, dst).reshape(NC, N_NODES, IN_DIM)
    out = _tc2(s2[0], s2[1], cnt[0], cnt[1], h, W2_r, b2.reshape(1, OUT_DIM))
    return out
